# baseline (speedup 1.0000x reference)
import jax
import jax.numpy as jnp
from jax import lax
from jax.experimental import pallas as pl
from jax.experimental.pallas import tpu as pltpu
from jax.experimental.pallas import tpu_sc as plsc
from functools import partial

B, N, S = 8, 4096, 1024
C1, C2 = 128, 256
M1, M2 = 256, 128
NBLK = 2048
NB = N // NBLK
P = B * N
CNT = float(P)
EPS = 1e-5

NW = 32
PW = P // NW
CP = 64
NCH = PW // CP


def _pass_gt(feats2t_ref, w1at_ref, gt_ref):
    gt_ref[...] = jnp.dot(feats2t_ref[0], w1at_ref[...],
                          preferred_element_type=jnp.float32)


def _pass_sel(xyz1_ref, xyz2t_ref, idx_ref, w_ref):
    b_i = pl.program_id(0)

    aq = xyz1_ref[0]
    bb = xyz2t_ref[0]
    a2 = jnp.sum(aq * aq, axis=1)
    b2 = jnp.sum(bb * bb, axis=0)
    abn = jnp.dot(aq.astype(jnp.bfloat16),
                  (-2.0 * bb).astype(jnp.bfloat16),
                  preferred_element_type=jnp.float32)
    d2 = (a2[:, None] + b2[None, :]) + abn

    iota1 = lax.broadcasted_iota(jnp.int32, (NBLK, S), 1)
    idxs, ws = [], []
    for k in range(3):
        m2 = jnp.min(d2, axis=1)
        sel = d2 == m2[:, None]
        idx = jnp.min(jnp.where(sel, iota1, S), axis=1)
        idxs.append(idx)
        ws.append(jnp.minimum(lax.rsqrt(jnp.maximum(m2, 0.0)), 1e8))
        if k < 2:
            d2 = jnp.where(iota1 == idx[:, None], jnp.float32(jnp.inf), d2)

    rws = 1.0 / (ws[0] + ws[1] + ws[2])
    for k in range(3):
        idx_ref[k] = idxs[k] + b_i * S
        wn = ws[k] * rws
        w_ref[k] = jnp.broadcast_to(wn[:, None], (NBLK, 16))


def _sc_body(gt_ref, idx_ref, wv_ref, out_ref,
             i0_v, i1_v, i2_v, r0_v, r1_v, r2_v,
             w0_v, w1_v, w2_v, o_v, sem):
    wid = lax.axis_index("s") * 2 + lax.axis_index("c")
    base0 = wid * PW

    def chunk(c, carry):
        base = base0 + c * CP
        pltpu.sync_copy(idx_ref.at[0, pl.ds(base, CP)], i0_v)
        pltpu.sync_copy(idx_ref.at[1, pl.ds(base, CP)], i1_v)
        pltpu.sync_copy(idx_ref.at[2, pl.ds(base, CP)], i2_v)
        pltpu.sync_copy(wv_ref.at[0, pl.ds(base, CP), :], w0_v)
        pltpu.sync_copy(wv_ref.at[1, pl.ds(base, CP), :], w1_v)
        pltpu.sync_copy(wv_ref.at[2, pl.ds(base, CP), :], w2_v)
        pltpu.async_copy(gt_ref.at[i0_v], r0_v, sem).wait()
        pltpu.async_copy(gt_ref.at[i1_v], r1_v, sem).wait()
        pltpu.async_copy(gt_ref.at[i2_v], r2_v, sem).wait()

        def point(p, cc):
            w0 = w0_v[p, :]
            w1 = w1_v[p, :]
            w2 = w2_v[p, :]
            for d in range(M1 // 16):
                sl = pl.ds(d * 16, 16)
                o_v[p, sl] = (r0_v[p, sl] * w0 + r1_v[p, sl] * w1
                              + r2_v[p, sl] * w2)
            return cc

        lax.fori_loop(0, CP, point, 0)
        pltpu.sync_copy(o_v, out_ref.at[pl.ds(base, CP), :])
        return carry

    lax.fori_loop(0, NCH, chunk, 0)


def _sc_gather(gt, idx3, w3):
    mesh = plsc.VectorSubcoreMesh(core_axis_name="c", subcore_axis_name="s")
    fn = partial(
        pl.kernel,
        mesh=mesh,
        out_type=jax.ShapeDtypeStruct((P, M1), jnp.float32),
        scratch_types=[
            pltpu.VMEM((CP,), jnp.int32),
            pltpu.VMEM((CP,), jnp.int32),
            pltpu.VMEM((CP,), jnp.int32),
            pltpu.VMEM((CP, M1), jnp.float32),
            pltpu.VMEM((CP, M1), jnp.float32),
            pltpu.VMEM((CP, M1), jnp.float32),
            pltpu.VMEM((CP, 16), jnp.float32),
            pltpu.VMEM((CP, 16), jnp.float32),
            pltpu.VMEM((CP, 16), jnp.float32),
            pltpu.VMEM((CP, M1), jnp.float32),
            pltpu.SemaphoreType.DMA,
        ],
    )(_sc_body)
    return fn(gt, idx3, w3)


def _pass_asm(interp_ref, feats1_ref, w1b_ref, b1_ref,
              y1_ref, s1_ref, ss1_ref):
    b_i = pl.program_id(0)
    n_i = pl.program_id(1)
    it = interp_ref[...]
    y = (jnp.transpose(it)
         + jnp.dot(w1b_ref[...], feats1_ref[0],
                   preferred_element_type=jnp.float32)
         + b1_ref[0][:, None])
    y1_ref[0] = y

    ps = jnp.sum(y, axis=1)
    pss = jnp.sum(y * y, axis=1)
    first = jnp.logical_and(b_i == 0, n_i == 0)

    @pl.when(first)
    def _():
        s1_ref[0] = ps
        ss1_ref[0] = pss

    @pl.when(jnp.logical_not(first))
    def _():
        s1_ref[0] += ps
        ss1_ref[0] += pss


def _pass_b(y1_ref, s1_ref, ss1_ref, g1_ref, be1_ref, w2_ref, b2_ref,
            y2_ref, s2_ref, ss2_ref):
    b_i = pl.program_id(0)
    n_i = pl.program_id(1)
    mean = s1_ref[0] / CNT
    var = ss1_ref[0] / CNT - mean * mean
    rstd = lax.rsqrt(var + EPS)
    scale = rstd * g1_ref[0]
    shift = be1_ref[0] - mean * scale
    z = jnp.maximum(y1_ref[0] * scale[:, None] + shift[:, None], 0.0)
    y = (jnp.dot(w2_ref[...], z, preferred_element_type=jnp.float32)
         + b2_ref[0][:, None])
    y2_ref[0] = y

    ps = jnp.sum(y, axis=1)
    pss = jnp.sum(y * y, axis=1)
    first = jnp.logical_and(b_i == 0, n_i == 0)

    @pl.when(first)
    def _():
        s2_ref[0] = ps
        ss2_ref[0] = pss

    @pl.when(jnp.logical_not(first))
    def _():
        s2_ref[0] += ps
        ss2_ref[0] += pss


def _pass_c(y2_ref, s2_ref, ss2_ref, g2_ref, be2_ref, out_ref):
    mean = s2_ref[0] / CNT
    var = ss2_ref[0] / CNT - mean * mean
    rstd = lax.rsqrt(var + EPS)
    scale = rstd * g2_ref[0]
    shift = be2_ref[0] - mean * scale
    out_ref[0] = jnp.maximum(y2_ref[0] * scale[:, None] + shift[:, None], 0.0)


def kernel(xyz1, xyz2, feats1, feats2, W1, b1, g1, be1, W2, b2, g2, be2):
    xyz2t = jnp.transpose(xyz2, (0, 2, 1))
    f2t = jnp.transpose(feats2, (0, 2, 1))
    w1at = jnp.transpose(W1[:, :C2])
    w1b = W1[:, C2:]
    b1r = b1.reshape(1, M1)
    g1r = g1.reshape(1, M1)
    be1r = be1.reshape(1, M1)
    b2r = b2.reshape(1, M2)
    g2r = g2.reshape(1, M2)
    be2r = be2.reshape(1, M2)

    gt = pl.pallas_call(
        _pass_gt,
        grid=(B,),
        in_specs=[
            pl.BlockSpec((1, S, C2), lambda b: (b, 0, 0)),
            pl.BlockSpec((C2, M1), lambda b: (0, 0)),
        ],
        out_specs=pl.BlockSpec((S, M1), lambda b: (b, 0)),
        out_shape=jax.ShapeDtypeStruct((B * S, M1), jnp.float32),
    )(f2t, w1at)

    idx3, w3 = pl.pallas_call(
        _pass_sel,
        grid=(B, NB),
        in_specs=[
            pl.BlockSpec((1, NBLK, 3), lambda b, n: (b, n, 0)),
            pl.BlockSpec((1, 3, S), lambda b, n: (b, 0, 0)),
        ],
        out_specs=[
            pl.BlockSpec((3, NBLK), lambda b, n: (0, b * NB + n)),
            pl.BlockSpec((3, NBLK, 16), lambda b, n: (0, b * NB + n, 0)),
        ],
        out_shape=[
            jax.ShapeDtypeStruct((3, P), jnp.int32),
            jax.ShapeDtypeStruct((3, P, 16), jnp.float32),
        ],
    )(xyz1, xyz2t)

    interp = _sc_gather(gt, idx3, w3)

    y1, s1, ss1 = pl.pallas_call(
        _pass_asm,
        grid=(B, NB),
        in_specs=[
            pl.BlockSpec((NBLK, M1), lambda b, n: (b * NB + n, 0)),
            pl.BlockSpec((1, C1, NBLK), lambda b, n: (b, 0, n)),
            pl.BlockSpec((M1, C1), lambda b, n: (0, 0)),
            pl.BlockSpec((1, M1), lambda b, n: (0, 0)),
        ],
        out_specs=[
            pl.BlockSpec((1, M1, NBLK), lambda b, n: (b, 0, n)),
            pl.BlockSpec((1, M1), lambda b, n: (0, 0)),
            pl.BlockSpec((1, M1), lambda b, n: (0, 0)),
        ],
        out_shape=[
            jax.ShapeDtypeStruct((B, M1, N), jnp.float32),
            jax.ShapeDtypeStruct((1, M1), jnp.float32),
            jax.ShapeDtypeStruct((1, M1), jnp.float32),
        ],
    )(interp, feats1, w1b, b1r)

    y2, s2, ss2 = pl.pallas_call(
        _pass_b,
        grid=(B, NB),
        in_specs=[
            pl.BlockSpec((1, M1, NBLK), lambda b, n: (b, 0, n)),
            pl.BlockSpec((1, M1), lambda b, n: (0, 0)),
            pl.BlockSpec((1, M1), lambda b, n: (0, 0)),
            pl.BlockSpec((1, M1), lambda b, n: (0, 0)),
            pl.BlockSpec((1, M1), lambda b, n: (0, 0)),
            pl.BlockSpec((M2, M1), lambda b, n: (0, 0)),
            pl.BlockSpec((1, M2), lambda b, n: (0, 0)),
        ],
        out_specs=[
            pl.BlockSpec((1, M2, NBLK), lambda b, n: (b, 0, n)),
            pl.BlockSpec((1, M2), lambda b, n: (0, 0)),
            pl.BlockSpec((1, M2), lambda b, n: (0, 0)),
        ],
        out_shape=[
            jax.ShapeDtypeStruct((B, M2, N), jnp.float32),
            jax.ShapeDtypeStruct((1, M2), jnp.float32),
            jax.ShapeDtypeStruct((1, M2), jnp.float32),
        ],
    )(y1, s1, ss1, g1r, be1r, W2, b2r)

    out = pl.pallas_call(
        _pass_c,
        grid=(B, NB),
        in_specs=[
            pl.BlockSpec((1, M2, NBLK), lambda b, n: (b, 0, n)),
            pl.BlockSpec((1, M2), lambda b, n: (0, 0)),
            pl.BlockSpec((1, M2), lambda b, n: (0, 0)),
            pl.BlockSpec((1, M2), lambda b, n: (0, 0)),
            pl.BlockSpec((1, M2), lambda b, n: (0, 0)),
        ],
        out_specs=pl.BlockSpec((1, M2, NBLK), lambda b, n: (b, 0, n)),
        out_shape=jax.ShapeDtypeStruct((B, M2, N), jnp.float32),
    )(y2, s2, ss2, g2r, be2r)

    return out


# speedup vs baseline: 1.0504x; 1.0504x over previous
import jax
import jax.numpy as jnp
from jax import lax
from jax.experimental import pallas as pl
from jax.experimental.pallas import tpu as pltpu
from jax.experimental.pallas import tpu_sc as plsc
from functools import partial

B, N, S = 8, 4096, 1024
C1, C2 = 128, 256
M1, M2 = 256, 128
NBLK = 2048
NB = N // NBLK
P = B * N
CNT = float(P)
EPS = 1e-5

NW = 32
PW = P // NW
CP = 64
NCH = PW // CP


def _pass_gt(feats2t_ref, w1at_ref, gt_ref):
    gt_ref[...] = jnp.dot(feats2t_ref[0], w1at_ref[...],
                          preferred_element_type=jnp.float32)


def _pass_sel(xyz1_ref, xyz2t_ref, idx_ref, w_ref):
    b_i = pl.program_id(0)

    aq = xyz1_ref[0]
    bb = xyz2t_ref[0]
    a2 = jnp.sum(aq * aq, axis=1)
    b2 = jnp.sum(bb * bb, axis=0)
    abn = jnp.dot(aq.astype(jnp.bfloat16),
                  (-2.0 * bb).astype(jnp.bfloat16),
                  preferred_element_type=jnp.float32)
    d2 = (a2[:, None] + b2[None, :]) + abn

    iota1 = lax.broadcasted_iota(jnp.int32, (NBLK, S), 1)
    idxs, ws = [], []
    for k in range(3):
        m2 = jnp.min(d2, axis=1)
        sel = d2 == m2[:, None]
        idx = jnp.min(jnp.where(sel, iota1, S), axis=1)
        idxs.append(idx)
        ws.append(jnp.minimum(lax.rsqrt(jnp.maximum(m2, 0.0)), 1e8))
        if k < 2:
            d2 = jnp.where(iota1 == idx[:, None], jnp.float32(jnp.inf), d2)

    rws = 1.0 / (ws[0] + ws[1] + ws[2])
    for k in range(3):
        idx_ref[k] = idxs[k] + b_i * S
        wn = ws[k] * rws
        w_ref[k] = jnp.broadcast_to(wn[:, None], (NBLK, 16))


def _sc_body(gt_ref, idx_ref, wv_ref, out_ref,
             i0_v, i1_v, i2_v, r0_v, r1_v, r2_v,
             w0_v, w1_v, w2_v, o_v, sem):
    wid = lax.axis_index("s") * 2 + lax.axis_index("c")
    base0 = wid * PW

    def chunk(c, carry):
        base = base0 + c * CP
        pltpu.sync_copy(idx_ref.at[0, pl.ds(base, CP)], i0_v)
        pltpu.sync_copy(idx_ref.at[1, pl.ds(base, CP)], i1_v)
        pltpu.sync_copy(idx_ref.at[2, pl.ds(base, CP)], i2_v)
        pltpu.sync_copy(wv_ref.at[0, pl.ds(base, CP), :], w0_v)
        pltpu.sync_copy(wv_ref.at[1, pl.ds(base, CP), :], w1_v)
        pltpu.sync_copy(wv_ref.at[2, pl.ds(base, CP), :], w2_v)
        c0 = pltpu.async_copy(gt_ref.at[i0_v], r0_v, sem)
        c1 = pltpu.async_copy(gt_ref.at[i1_v], r1_v, sem)
        c2 = pltpu.async_copy(gt_ref.at[i2_v], r2_v, sem)
        c0.wait()
        c1.wait()
        c2.wait()

        def point(p, cc):
            w0 = w0_v[p, :]
            w1 = w1_v[p, :]
            w2 = w2_v[p, :]
            for d in range(M1 // 16):
                sl = pl.ds(d * 16, 16)
                o_v[p, sl] = (r0_v[p, sl] * w0 + r1_v[p, sl] * w1
                              + r2_v[p, sl] * w2)
            return cc

        lax.fori_loop(0, CP, point, 0)
        pltpu.sync_copy(o_v, out_ref.at[pl.ds(base, CP), :])
        return carry

    lax.fori_loop(0, NCH, chunk, 0)


def _sc_gather(gt, idx3, w3):
    mesh = plsc.VectorSubcoreMesh(core_axis_name="c", subcore_axis_name="s")
    fn = partial(
        pl.kernel,
        mesh=mesh,
        out_type=jax.ShapeDtypeStruct((P, M1), jnp.float32),
        scratch_types=[
            pltpu.VMEM((CP,), jnp.int32),
            pltpu.VMEM((CP,), jnp.int32),
            pltpu.VMEM((CP,), jnp.int32),
            pltpu.VMEM((CP, M1), jnp.float32),
            pltpu.VMEM((CP, M1), jnp.float32),
            pltpu.VMEM((CP, M1), jnp.float32),
            pltpu.VMEM((CP, 16), jnp.float32),
            pltpu.VMEM((CP, 16), jnp.float32),
            pltpu.VMEM((CP, 16), jnp.float32),
            pltpu.VMEM((CP, M1), jnp.float32),
            pltpu.SemaphoreType.DMA,
        ],
    )(_sc_body)
    return fn(gt, idx3, w3)


def _pass_asm(interp_ref, feats1_ref, w1b_ref, b1_ref,
              y1_ref, s1_ref, ss1_ref):
    b_i = pl.program_id(0)
    n_i = pl.program_id(1)
    it = interp_ref[...]
    y = (jnp.transpose(it)
         + jnp.dot(w1b_ref[...], feats1_ref[0],
                   preferred_element_type=jnp.float32)
         + b1_ref[0][:, None])
    y1_ref[0] = y

    ps = jnp.sum(y, axis=1)
    pss = jnp.sum(y * y, axis=1)
    first = jnp.logical_and(b_i == 0, n_i == 0)

    @pl.when(first)
    def _():
        s1_ref[0] = ps
        ss1_ref[0] = pss

    @pl.when(jnp.logical_not(first))
    def _():
        s1_ref[0] += ps
        ss1_ref[0] += pss


def _pass_b(y1_ref, s1_ref, ss1_ref, g1_ref, be1_ref, w2_ref, b2_ref,
            y2_ref, s2_ref, ss2_ref):
    b_i = pl.program_id(0)
    n_i = pl.program_id(1)
    mean = s1_ref[0] / CNT
    var = ss1_ref[0] / CNT - mean * mean
    rstd = lax.rsqrt(var + EPS)
    scale = rstd * g1_ref[0]
    shift = be1_ref[0] - mean * scale
    z = jnp.maximum(y1_ref[0] * scale[:, None] + shift[:, None], 0.0)
    y = (jnp.dot(w2_ref[...], z, preferred_element_type=jnp.float32)
         + b2_ref[0][:, None])
    y2_ref[0] = y

    ps = jnp.sum(y, axis=1)
    pss = jnp.sum(y * y, axis=1)
    first = jnp.logical_and(b_i == 0, n_i == 0)

    @pl.when(first)
    def _():
        s2_ref[0] = ps
        ss2_ref[0] = pss

    @pl.when(jnp.logical_not(first))
    def _():
        s2_ref[0] += ps
        ss2_ref[0] += pss


def _pass_c(y2_ref, s2_ref, ss2_ref, g2_ref, be2_ref, out_ref):
    mean = s2_ref[0] / CNT
    var = ss2_ref[0] / CNT - mean * mean
    rstd = lax.rsqrt(var + EPS)
    scale = rstd * g2_ref[0]
    shift = be2_ref[0] - mean * scale
    out_ref[0] = jnp.maximum(y2_ref[0] * scale[:, None] + shift[:, None], 0.0)


def kernel(xyz1, xyz2, feats1, feats2, W1, b1, g1, be1, W2, b2, g2, be2):
    xyz2t = jnp.transpose(xyz2, (0, 2, 1))
    f2t = jnp.transpose(feats2, (0, 2, 1))
    w1at = jnp.transpose(W1[:, :C2])
    w1b = W1[:, C2:]
    b1r = b1.reshape(1, M1)
    g1r = g1.reshape(1, M1)
    be1r = be1.reshape(1, M1)
    b2r = b2.reshape(1, M2)
    g2r = g2.reshape(1, M2)
    be2r = be2.reshape(1, M2)

    gt = pl.pallas_call(
        _pass_gt,
        grid=(B,),
        in_specs=[
            pl.BlockSpec((1, S, C2), lambda b: (b, 0, 0)),
            pl.BlockSpec((C2, M1), lambda b: (0, 0)),
        ],
        out_specs=pl.BlockSpec((S, M1), lambda b: (b, 0)),
        out_shape=jax.ShapeDtypeStruct((B * S, M1), jnp.float32),
    )(f2t, w1at)

    idx3, w3 = pl.pallas_call(
        _pass_sel,
        grid=(B, NB),
        in_specs=[
            pl.BlockSpec((1, NBLK, 3), lambda b, n: (b, n, 0)),
            pl.BlockSpec((1, 3, S), lambda b, n: (b, 0, 0)),
        ],
        out_specs=[
            pl.BlockSpec((3, NBLK), lambda b, n: (0, b * NB + n)),
            pl.BlockSpec((3, NBLK, 16), lambda b, n: (0, b * NB + n, 0)),
        ],
        out_shape=[
            jax.ShapeDtypeStruct((3, P), jnp.int32),
            jax.ShapeDtypeStruct((3, P, 16), jnp.float32),
        ],
    )(xyz1, xyz2t)

    interp = _sc_gather(gt, idx3, w3)

    y1, s1, ss1 = pl.pallas_call(
        _pass_asm,
        grid=(B, NB),
        in_specs=[
            pl.BlockSpec((NBLK, M1), lambda b, n: (b * NB + n, 0)),
            pl.BlockSpec((1, C1, NBLK), lambda b, n: (b, 0, n)),
            pl.BlockSpec((M1, C1), lambda b, n: (0, 0)),
            pl.BlockSpec((1, M1), lambda b, n: (0, 0)),
        ],
        out_specs=[
            pl.BlockSpec((1, M1, NBLK), lambda b, n: (b, 0, n)),
            pl.BlockSpec((1, M1), lambda b, n: (0, 0)),
            pl.BlockSpec((1, M1), lambda b, n: (0, 0)),
        ],
        out_shape=[
            jax.ShapeDtypeStruct((B, M1, N), jnp.float32),
            jax.ShapeDtypeStruct((1, M1), jnp.float32),
            jax.ShapeDtypeStruct((1, M1), jnp.float32),
        ],
    )(interp, feats1, w1b, b1r)

    y2, s2, ss2 = pl.pallas_call(
        _pass_b,
        grid=(B, NB),
        in_specs=[
            pl.BlockSpec((1, M1, NBLK), lambda b, n: (b, 0, n)),
            pl.BlockSpec((1, M1), lambda b, n: (0, 0)),
            pl.BlockSpec((1, M1), lambda b, n: (0, 0)),
            pl.BlockSpec((1, M1), lambda b, n: (0, 0)),
            pl.BlockSpec((1, M1), lambda b, n: (0, 0)),
            pl.BlockSpec((M2, M1), lambda b, n: (0, 0)),
            pl.BlockSpec((1, M2), lambda b, n: (0, 0)),
        ],
        out_specs=[
            pl.BlockSpec((1, M2, NBLK), lambda b, n: (b, 0, n)),
            pl.BlockSpec((1, M2), lambda b, n: (0, 0)),
            pl.BlockSpec((1, M2), lambda b, n: (0, 0)),
        ],
        out_shape=[
            jax.ShapeDtypeStruct((B, M2, N), jnp.float32),
            jax.ShapeDtypeStruct((1, M2), jnp.float32),
            jax.ShapeDtypeStruct((1, M2), jnp.float32),
        ],
    )(y1, s1, ss1, g1r, be1r, W2, b2r)

    out = pl.pallas_call(
        _pass_c,
        grid=(B, NB),
        in_specs=[
            pl.BlockSpec((1, M2, NBLK), lambda b, n: (b, 0, n)),
            pl.BlockSpec((1, M2), lambda b, n: (0, 0)),
            pl.BlockSpec((1, M2), lambda b, n: (0, 0)),
            pl.BlockSpec((1, M2), lambda b, n: (0, 0)),
            pl.BlockSpec((1, M2), lambda b, n: (0, 0)),
        ],
        out_specs=pl.BlockSpec((1, M2, NBLK), lambda b, n: (b, 0, n)),
        out_shape=jax.ShapeDtypeStruct((B, M2, N), jnp.float32),
    )(y2, s2, ss2, g2r, be2r)

    return out


# speedup vs baseline: 2.3895x; 2.2749x over previous
import jax
import jax.numpy as jnp
from jax import lax
from jax.experimental import pallas as pl
from jax.experimental.pallas import tpu as pltpu
from functools import partial

B, N, S = 8, 4096, 1024
C1, C2 = 128, 256
M1, M2 = 256, 128
NBLK = 2048
NB = N // NBLK
CNT = float(B * N)
EPS = 1e-5


def _pass_a(xyz1_ref, xyz2t_ref, feats2_ref, feats1_ref, w1a_ref, w1b_ref,
            b1_ref, y1_ref, s1_ref, ss1_ref, g_ref, wmat_ref):
    b_i = pl.program_id(0)
    n_i = pl.program_id(1)

    @pl.when(n_i == 0)
    def _():
        g_ref[...] = jnp.dot(w1a_ref[...], feats2_ref[0],
                             preferred_element_type=jnp.float32)

    aq = xyz1_ref[0]
    bb = xyz2t_ref[0]
    a2 = jnp.sum(aq * aq, axis=1)
    b2 = jnp.sum(bb * bb, axis=0)
    abn = jnp.dot(aq.astype(jnp.bfloat16),
                  (-2.0 * bb).astype(jnp.bfloat16),
                  preferred_element_type=jnp.float32)
    t = b2[None, :] + abn

    sels, ws = [], []
    dd = t
    for k in range(3):
        mt = jnp.min(dd, axis=1)
        sel = dd == mt[:, None]
        sels.append(sel)
        m2 = a2 + mt
        ws.append(jnp.minimum(lax.rsqrt(jnp.maximum(m2, 0.0)), 1e8))
        if k < 2:
            dd = jnp.where(sel, jnp.float32(jnp.inf), dd)

    rws = 1.0 / (ws[0] + ws[1] + ws[2])
    wm = (jnp.where(sels[0], (ws[0] * rws)[:, None], 0.0)
          + jnp.where(sels[1], (ws[1] * rws)[:, None], 0.0)
          + jnp.where(sels[2], (ws[2] * rws)[:, None], 0.0))
    wmat_ref[...] = wm

    rs = jnp.sum(wm, axis=1)
    bad = jnp.max(jnp.abs(rs - 1.0))

    @pl.when(bad > 1e-4)
    def _():
        iota1 = lax.broadcasted_iota(jnp.int32, (NBLK, S), 1)
        de = (a2[:, None] + b2[None, :]) + abn
        emasks, ews = [], []
        for k in range(3):
            em2 = jnp.min(de, axis=1)
            esel = de == em2[:, None]
            eidx = jnp.min(jnp.where(esel, iota1, S), axis=1)
            edk = jnp.sqrt(jnp.maximum(em2, 0.0))
            ews.append(1.0 / jnp.maximum(edk, 1e-8))
            emk = iota1 == eidx[:, None]
            emasks.append(emk)
            if k < 2:
                de = jnp.where(emk, jnp.float32(jnp.inf), de)
        ewsum = ews[0] + ews[1] + ews[2]
        wmat_ref[...] = (
            jnp.where(emasks[0], (ews[0] / ewsum)[:, None], 0.0)
            + jnp.where(emasks[1], (ews[1] / ewsum)[:, None], 0.0)
            + jnp.where(emasks[2], (ews[2] / ewsum)[:, None], 0.0))

    y = (lax.dot_general(g_ref[...], wmat_ref[...], (((1,), (1,)), ((), ())),
                         preferred_element_type=jnp.float32)
         + jnp.dot(w1b_ref[...], feats1_ref[0],
                   preferred_element_type=jnp.float32)
         + b1_ref[0][:, None])
    y1_ref[0] = y

    ps = jnp.sum(y, axis=1)
    pss = jnp.sum(y * y, axis=1)
    first = jnp.logical_and(b_i == 0, n_i == 0)

    @pl.when(first)
    def _():
        s1_ref[0] = ps
        ss1_ref[0] = pss

    @pl.when(jnp.logical_not(first))
    def _():
        s1_ref[0] += ps
        ss1_ref[0] += pss


def _pass_b(y1_ref, s1_ref, ss1_ref, g1_ref, be1_ref, w2_ref, b2_ref,
            y2_ref, s2_ref, ss2_ref):
    b_i = pl.program_id(0)
    n_i = pl.program_id(1)
    mean = s1_ref[0] / CNT
    var = ss1_ref[0] / CNT - mean * mean
    rstd = lax.rsqrt(var + EPS)
    scale = rstd * g1_ref[0]
    shift = be1_ref[0] - mean * scale
    z = jnp.maximum(y1_ref[0] * scale[:, None] + shift[:, None], 0.0)
    y = (jnp.dot(w2_ref[...], z, preferred_element_type=jnp.float32)
         + b2_ref[0][:, None])
    y2_ref[0] = y

    ps = jnp.sum(y, axis=1)
    pss = jnp.sum(y * y, axis=1)
    first = jnp.logical_and(b_i == 0, n_i == 0)

    @pl.when(first)
    def _():
        s2_ref[0] = ps
        ss2_ref[0] = pss

    @pl.when(jnp.logical_not(first))
    def _():
        s2_ref[0] += ps
        ss2_ref[0] += pss


def _pass_c(y2_ref, s2_ref, ss2_ref, g2_ref, be2_ref, out_ref):
    mean = s2_ref[0] / CNT
    var = ss2_ref[0] / CNT - mean * mean
    rstd = lax.rsqrt(var + EPS)
    scale = rstd * g2_ref[0]
    shift = be2_ref[0] - mean * scale
    out_ref[0] = jnp.maximum(y2_ref[0] * scale[:, None] + shift[:, None], 0.0)


def kernel(xyz1, xyz2, feats1, feats2, W1, b1, g1, be1, W2, b2, g2, be2):
    xyz2t = jnp.transpose(xyz2, (0, 2, 1))
    w1a = W1[:, :C2]
    w1b = W1[:, C2:]
    b1r = b1.reshape(1, M1)
    g1r = g1.reshape(1, M1)
    be1r = be1.reshape(1, M1)
    b2r = b2.reshape(1, M2)
    g2r = g2.reshape(1, M2)
    be2r = be2.reshape(1, M2)

    y1, s1, ss1 = pl.pallas_call(
        _pass_a,
        grid=(B, NB),
        in_specs=[
            pl.BlockSpec((1, NBLK, 3), lambda b, n: (b, n, 0)),
            pl.BlockSpec((1, 3, S), lambda b, n: (b, 0, 0)),
            pl.BlockSpec((1, C2, S), lambda b, n: (b, 0, 0)),
            pl.BlockSpec((1, C1, NBLK), lambda b, n: (b, 0, n)),
            pl.BlockSpec((M1, C2), lambda b, n: (0, 0)),
            pl.BlockSpec((M1, C1), lambda b, n: (0, 0)),
            pl.BlockSpec((1, M1), lambda b, n: (0, 0)),
        ],
        out_specs=[
            pl.BlockSpec((1, M1, NBLK), lambda b, n: (b, 0, n)),
            pl.BlockSpec((1, M1), lambda b, n: (0, 0)),
            pl.BlockSpec((1, M1), lambda b, n: (0, 0)),
        ],
        out_shape=[
            jax.ShapeDtypeStruct((B, M1, N), jnp.float32),
            jax.ShapeDtypeStruct((1, M1), jnp.float32),
            jax.ShapeDtypeStruct((1, M1), jnp.float32),
        ],
        scratch_shapes=[pltpu.VMEM((M1, S), jnp.float32),
                        pltpu.VMEM((NBLK, S), jnp.float32)],
    )(xyz1, xyz2t, feats2, feats1, w1a, w1b, b1r)

    y2, s2, ss2 = pl.pallas_call(
        _pass_b,
        grid=(B, NB),
        in_specs=[
            pl.BlockSpec((1, M1, NBLK), lambda b, n: (b, 0, n)),
            pl.BlockSpec((1, M1), lambda b, n: (0, 0)),
            pl.BlockSpec((1, M1), lambda b, n: (0, 0)),
            pl.BlockSpec((1, M1), lambda b, n: (0, 0)),
            pl.BlockSpec((1, M1), lambda b, n: (0, 0)),
            pl.BlockSpec((M2, M1), lambda b, n: (0, 0)),
            pl.BlockSpec((1, M2), lambda b, n: (0, 0)),
        ],
        out_specs=[
            pl.BlockSpec((1, M2, NBLK), lambda b, n: (b, 0, n)),
            pl.BlockSpec((1, M2), lambda b, n: (0, 0)),
            pl.BlockSpec((1, M2), lambda b, n: (0, 0)),
        ],
        out_shape=[
            jax.ShapeDtypeStruct((B, M2, N), jnp.float32),
            jax.ShapeDtypeStruct((1, M2), jnp.float32),
            jax.ShapeDtypeStruct((1, M2), jnp.float32),
        ],
    )(y1, s1, ss1, g1r, be1r, W2, b2r)

    out = pl.pallas_call(
        _pass_c,
        grid=(B, NB),
        in_specs=[
            pl.BlockSpec((1, M2, NBLK), lambda b, n: (b, 0, n)),
            pl.BlockSpec((1, M2), lambda b, n: (0, 0)),
            pl.BlockSpec((1, M2), lambda b, n: (0, 0)),
            pl.BlockSpec((1, M2), lambda b, n: (0, 0)),
            pl.BlockSpec((1, M2), lambda b, n: (0, 0)),
        ],
        out_specs=pl.BlockSpec((1, M2, NBLK), lambda b, n: (b, 0, n)),
        out_shape=jax.ShapeDtypeStruct((B, M2, N), jnp.float32),
    )(y2, s2, ss2, g2r, be2r)

    return out
